# baseline (speedup 1.0000x reference)
import functools

import jax
import jax.numpy as jnp
from jax.experimental import pallas as pl
from jax.experimental.pallas import tpu as pltpu

_F32 = jnp.float32
_BF16 = jnp.bfloat16
_NEG = -9e15


def _dot(a, b):
    return jnp.dot(a, b, preferred_element_type=_F32)


def _dg(a, b, dims):
    return jax.lax.dot_general(a, b, (dims, ((), ())),
                               preferred_element_type=_F32)


def _prep_body(hid, kb, x_ref, w_ref, a_ref, fcw_ref, fcb_ref, cb_ref,
               wh_ref, wh1_ref, wh2r_ref, f0_ref, lsum_ref):
    xb = x_ref[...]
    whb = _dot(xb, w_ref[...])
    wh_ref[...] = whb
    a1 = a_ref[:hid, :]
    a2 = a_ref[hid:, :]
    wh1_ref[...] = _dot(whb, a1)
    wh2r_ref[...] = _dg(a2, whb, (((0,), (1,))))
    f0_ref[...] = _dg(whb, fcw_ref[...], (((1,), (1,)))) + fcb_ref[...]
    cb = cb_ref[...]
    xsq = jnp.sum(xb * xb, axis=1, keepdims=True)
    cbsq = _dg(jnp.ones((1, cb.shape[1]), _F32), cb * cb,
               (((1,), (1,))))
    cross = _dg(xb, cb, (((1,), (1,))))
    d2 = jnp.maximum(xsq - 2.0 * cross + cbsq, 0.0)
    d = jnp.sqrt(d2)
    s = jnp.exp(-d) + 1e-10
    lsum_ref[...] = (jnp.sum(jnp.log(s), axis=1, keepdims=True)
                     - kb * jnp.log(jnp.sum(s, axis=1, keepdims=True)))


def _fused_body(c, n, r, sup_ref, wh1_ref, wh2r_ref, wh_ref, gcnw_ref,
                y_ref, lsum_ref, q_ref, f0_ref, res_ref,
                z0_s, xw_s, outb_s, f0b_s, g_s, m_s, cs_s, cnt_s, p_s,
                sm_s, supb_s):
    s = pl.program_id(0)
    i = pl.program_id(1)
    rows = pl.ds(i * r, r)

    @pl.when(s == 0)
    def _stage0():
        sup = sup_ref[...]
        supb_s[rows, :] = sup.astype(_BF16)
        e = wh1_ref[...] + wh2r_ref[...]
        e = jnp.where(e >= 0, e, 0.2 * e)
        logits = jnp.where(sup > 0, e, _NEG)
        mx = jnp.max(logits, axis=1, keepdims=True)
        p = jnp.exp(logits - mx)
        sm = jnp.sum(p, axis=1, keepdims=True)
        p_s[rows, :] = p.astype(_BF16)
        sm_s[rows, :] = sm
        hp = _dot(p, wh_ref[...]) / sm
        z0_s[rows, :] = hp
        gat = jnp.where(hp > 0, hp, jnp.exp(jnp.minimum(hp, 0.0)) - 1.0)
        xw_s[rows, :] = _dot(gat, gcnw_ref[...]).astype(_BF16)
        yrow = y_ref[0]
        oh = (jax.lax.broadcasted_iota(jnp.int32, (c, r), 0) == yrow)
        oh = oh.astype(_F32)
        csb = _dot(oh, hp)
        cntb = _dot(oh, jnp.ones((r, cs_s.shape[1]), dtype=_F32))

        @pl.when(i == 0)
        def _init():
            cs_s[...] = csb
            cnt_s[...] = cntb

        @pl.when(i > 0)
        def _acc():
            cs_s[...] += csb
            cnt_s[...] += cntb

    @pl.when(s == 1)
    def _stage1():
        out_blk = _dot(supb_s[rows, :], xw_s[...])
        outb_s[rows, :] = out_blk.astype(_BF16)
        f0_blk = f0_ref[rows, :]
        f0b_s[rows, :] = f0_blk.astype(_BF16)
        gb = _dg(f0_blk, out_blk, (((0,), (0,))))
        mb = _dg(f0_blk, f0_blk, (((0,), (0,))))

        @pl.when(i == 0)
        def _init():
            g_s[...] = gb
            m_s[...] = mb

        @pl.when(i > 0)
        def _acc():
            g_s[...] += gb
            m_s[...] += mb

    @pl.when(s == 2)
    def _stage2():
        ca = cs_s[...] / jnp.maximum(cnt_s[...], 1.0)
        z0 = z0_s[rows, :]
        yrow = y_ref[0]
        oh = (jax.lax.broadcasted_iota(jnp.int32, (c, r), 0) == yrow)
        oh = oh.astype(_F32)
        logq = jnp.log(q_ref[...])
        d_list, s_list = [], []
        ssum = None
        for k in range(c):
            diff = z0 - ca[k:k + 1, :]
            d = jnp.sqrt(jnp.sum(diff * diff, axis=1, keepdims=True))
            sk = jnp.exp(-d) + 1e-10
            d_list.append(d)
            s_list.append(sk)
            ssum = sk if ssum is None else ssum + sk
        lsum = lsum_ref[...]
        phi_list, db_list = [], []
        dsum = None
        for k in range(c):
            phi = s_list[k] / ssum
            sylq = _dg(oh, logq[:, k:k + 1], (((0,), (0,))))
            db = phi * lsum - sylq
            phi_list.append(phi)
            db_list.append(db)
            dsum = db if dsum is None else dsum + db
        csum = None
        vc = None
        for k in range(c):
            cmat = (dsum * phi_list[k] - db_list[k]) / (d_list[k] + 1e-12)
            csum = cmat if csum is None else csum + cmat
            term = cmat * ca[k:k + 1, :]
            vc = term if vc is None else vc + term
        v = csum * z0 - vc
        pb = p_s[rows, :]
        pf = pb.astype(_F32)
        inv_sm = 1.0 / sm_s[rows, :]
        att_f0 = _dot(pb, f0b_s[...]) * inv_sm
        cross = jnp.sum(v * att_f0, axis=1, keepdims=True) * (1.0 / n)
        asq = jnp.sum(pf * pf, axis=1, keepdims=True) * (inv_sm * inv_sm)
        vm = _dot(v, m_s[...])
        gsq = jnp.sum(vm * v, axis=1, keepdims=True) * (1.0 / (n * n))
        rn = jnp.sqrt(asq - 2.0 * cross + gsq)
        pout = _dot(pb, outb_s[...]) * inv_sm
        z1 = (pout - _dot(v, g_s[...]) * (1.0 / n)) / rn
        res_ref[...] = jnp.maximum(z1, 0.0)


def kernel(x, support, y, C_b_prime, Q, W, a, fc0_W, fc0_b, gcn_W):
    n, d_in = x.shape
    hid = W.shape[1]
    d_out = gcn_W.shape[1]
    c = Q.shape[0]
    kb = C_b_prime.shape[0]
    r = 128
    nblk = n // r
    y3 = y.astype(jnp.int32).reshape(nblk, 1, r)
    fcb = fc0_b.reshape(1, hid)

    row = lambda bs: pl.BlockSpec(bs, lambda i: (i, 0))
    full = lambda bs: pl.BlockSpec(bs, lambda i: (0, 0))

    wh, wh1, wh2r, f0, lsum = pl.pallas_call(
        functools.partial(_prep_body, hid, kb),
        grid=(nblk,),
        in_specs=[row((r, d_in)), full((d_in, hid)), full((2 * hid, 1)),
                  full((hid, hid)), full((1, hid)), full((kb, d_in))],
        out_specs=[row((r, hid)), row((r, 1)),
                   pl.BlockSpec((1, r), lambda i: (0, i)),
                   row((r, hid)), row((r, 1))],
        out_shape=[jax.ShapeDtypeStruct((n, hid), _F32),
                   jax.ShapeDtypeStruct((n, 1), _F32),
                   jax.ShapeDtypeStruct((1, n), _F32),
                   jax.ShapeDtypeStruct((n, hid), _F32),
                   jax.ShapeDtypeStruct((n, 1), _F32)],
    )(x, W, a, fc0_W, fcb, C_b_prime)

    rowf = lambda bs: pl.BlockSpec(bs, lambda s, i: (i, 0))
    fullf = lambda bs: pl.BlockSpec(bs, lambda s, i: (0, 0))
    sup_spec = pl.BlockSpec(
        (r, n), lambda s, i: (jnp.where(s == 0, i, nblk - 1), 0))

    res = pl.pallas_call(
        functools.partial(_fused_body, c, n, r),
        grid=(3, nblk),
        in_specs=[sup_spec, rowf((r, 1)), fullf((1, n)), fullf((n, hid)),
                  fullf((hid, d_out)),
                  pl.BlockSpec((1, 1, r), lambda s, i: (i, 0, 0)),
                  rowf((r, 1)), fullf((c, c)), fullf((n, hid))],
        out_specs=pl.BlockSpec(
            (r, d_out), lambda s, i: (jnp.where(s == 2, i, 0), 0)),
        out_shape=jax.ShapeDtypeStruct((n, d_out), _F32),
        scratch_shapes=[pltpu.VMEM((n, hid), _F32),
                        pltpu.VMEM((n, d_out), _BF16),
                        pltpu.VMEM((n, d_out), _BF16),
                        pltpu.VMEM((n, hid), _BF16),
                        pltpu.VMEM((hid, d_out), _F32),
                        pltpu.VMEM((hid, hid), _F32),
                        pltpu.VMEM((c, hid), _F32),
                        pltpu.VMEM((c, hid), _F32),
                        pltpu.VMEM((n, n), _BF16),
                        pltpu.VMEM((n, 1), _F32),
                        pltpu.VMEM((n, n), _BF16)],
    )(support, wh1, wh2r, wh, gcn_W, y3, lsum, Q, f0)
    return res


# speedup vs baseline: 1.1016x; 1.1016x over previous
import functools

import jax
import jax.numpy as jnp
from jax.experimental import pallas as pl
from jax.experimental.pallas import tpu as pltpu

_F32 = jnp.float32
_BF16 = jnp.bfloat16


def _dot(a, b):
    return jnp.dot(a, b, preferred_element_type=_F32)


def _dg(a, b, dims):
    return jax.lax.dot_general(a, b, (dims, ((), ())),
                               preferred_element_type=_F32)


def _body(c, n, r, hid, kb,
          x_ref, sup_ref, w_ref, a_ref, fcw_ref, fcb_ref, cb_ref,
          gcnw_ref, y_ref, q_ref, res_ref,
          whb_s, wh1_s, wh2r_s, lsum_s, f0_s, f0b_s, z0_s, xw_s, outb_s,
          g_s, m_s, cs_s, cnt_s, p_s, sm_s, psq_s, supb_s):
    s = pl.program_id(0)
    i = pl.program_id(1)
    rows = pl.ds(i * r, r)

    @pl.when(s == 0)
    def _prep():
        xb = x_ref[...]
        whb = _dot(xb, w_ref[...])
        whb_s[rows, :] = whb.astype(_BF16)
        a1 = a_ref[:hid, :]
        a2 = a_ref[hid:, :]
        wh1_s[rows, :] = _dot(whb, a1)
        wh2r_s[:, rows] = _dg(a2, whb, (((0,), (1,))))
        f0b = _dg(whb, fcw_ref[...], (((1,), (1,)))) + fcb_ref[...]
        f0_s[rows, :] = f0b
        f0b_s[rows, :] = f0b.astype(_BF16)
        cb = cb_ref[...]
        xsq = jnp.sum(xb * xb, axis=1, keepdims=True)
        cbsq = _dg(jnp.ones((1, cb.shape[1]), _F32), cb * cb,
                   (((1,), (1,))))
        cross = _dg(xb, cb, (((1,), (1,))))
        d = jnp.sqrt(jnp.maximum(xsq - 2.0 * cross + cbsq, 0.0))
        sx = jnp.exp(-d) + 1e-10
        lsum_s[rows, :] = (jnp.sum(jnp.log(sx), axis=1, keepdims=True)
                           - kb * jnp.log(jnp.sum(sx, axis=1, keepdims=True)))

    @pl.when(s == 1)
    def _attn():
        sup = sup_ref[...]
        supb_s[rows, :] = sup.astype(_BF16)
        e = wh1_s[rows, :] + wh2r_s[...]
        e = jnp.maximum(e, 0.2 * e)
        p = jnp.where(sup > 0, jnp.exp(e), 0.0)
        pb = p.astype(_BF16)
        p_s[rows, :] = pb
        sm = jnp.sum(p, axis=1, keepdims=True)
        sm_s[rows, :] = sm
        psq_s[rows, :] = jnp.sum(p * p, axis=1, keepdims=True)
        hp = _dot(pb, whb_s[...]) / sm
        z0_s[rows, :] = hp
        gat = jnp.where(hp > 0, hp, jnp.exp(jnp.minimum(hp, 0.0)) - 1.0)
        xw_s[rows, :] = _dot(gat.astype(_BF16),
                             gcnw_ref[...].astype(_BF16)).astype(_BF16)
        yrow = y_ref[0]
        oh = (jax.lax.broadcasted_iota(jnp.int32, (c, r), 0) == yrow)
        oh = oh.astype(_F32)
        csb = _dot(oh, hp)
        cntb = _dot(oh, jnp.ones((r, cs_s.shape[1]), dtype=_F32))

        @pl.when(i == 0)
        def _init():
            cs_s[...] = csb
            cnt_s[...] = cntb

        @pl.when(i > 0)
        def _acc():
            cs_s[...] += csb
            cnt_s[...] += cntb

    @pl.when(s == 2)
    def _spread():
        out_blk = _dot(supb_s[rows, :], xw_s[...])
        outb_s[rows, :] = out_blk.astype(_BF16)
        f0_blk = f0_s[rows, :]
        gb = _dg(f0_blk, out_blk, (((0,), (0,))))
        mb = _dg(f0_blk, f0_blk, (((0,), (0,))))

        @pl.when(i == 0)
        def _init():
            g_s[...] = gb
            m_s[...] = mb

        @pl.when(i > 0)
        def _acc():
            g_s[...] += gb
            m_s[...] += mb

    @pl.when(s == 3)
    def _final():
        ca = cs_s[...] / jnp.maximum(cnt_s[...], 1.0)
        z0 = z0_s[rows, :]
        yrow = y_ref[0]
        oh = (jax.lax.broadcasted_iota(jnp.int32, (c, r), 0) == yrow)
        oh = oh.astype(_F32)
        logq = jnp.log(q_ref[...])
        d_list, s_list = [], []
        ssum = None
        for k in range(c):
            diff = z0 - ca[k:k + 1, :]
            d = jnp.sqrt(jnp.sum(diff * diff, axis=1, keepdims=True))
            sk = jnp.exp(-d) + 1e-10
            d_list.append(d)
            s_list.append(sk)
            ssum = sk if ssum is None else ssum + sk
        lsum = lsum_s[rows, :]
        phi_list, db_list = [], []
        dsum = None
        for k in range(c):
            phi = s_list[k] / ssum
            sylq = _dg(oh, logq[:, k:k + 1], (((0,), (0,))))
            db = phi * lsum - sylq
            phi_list.append(phi)
            db_list.append(db)
            dsum = db if dsum is None else dsum + db
        csum = None
        vc = None
        for k in range(c):
            cmat = (dsum * phi_list[k] - db_list[k]) / (d_list[k] + 1e-12)
            csum = cmat if csum is None else csum + cmat
            term = cmat * ca[k:k + 1, :]
            vc = term if vc is None else vc + term
        v = csum * z0 - vc
        pb = p_s[rows, :]
        inv_sm = 1.0 / sm_s[rows, :]
        att_f0 = _dot(pb, f0b_s[...]) * inv_sm
        cross = jnp.sum(v * att_f0, axis=1, keepdims=True) * (1.0 / n)
        asq = psq_s[rows, :] * (inv_sm * inv_sm)
        vm = _dot(v, m_s[...])
        gsq = jnp.sum(vm * v, axis=1, keepdims=True) * (1.0 / (n * n))
        rn = jnp.sqrt(asq - 2.0 * cross + gsq)
        pout = _dot(pb, outb_s[...]) * inv_sm
        z1 = (pout - _dot(v, g_s[...]) * (1.0 / n)) / rn
        res_ref[...] = jnp.maximum(z1, 0.0)


def kernel(x, support, y, C_b_prime, Q, W, a, fc0_W, fc0_b, gcn_W):
    n, d_in = x.shape
    hid = W.shape[1]
    d_out = gcn_W.shape[1]
    c = Q.shape[0]
    kb = C_b_prime.shape[0]
    r = 128
    nblk = n // r
    y3 = y.astype(jnp.int32).reshape(nblk, 1, r)
    fcb = fc0_b.reshape(1, hid)

    full = lambda bs: pl.BlockSpec(bs, lambda s, i: (0, 0))
    x_spec = pl.BlockSpec(
        (r, d_in), lambda s, i: (jnp.where(s == 0, i, nblk - 1), 0))
    sup_spec = pl.BlockSpec(
        (r, n),
        lambda s, i: (jnp.where(s == 1, i,
                                jnp.where(s == 0, 0, nblk - 1)), 0))

    res = pl.pallas_call(
        functools.partial(_body, c, n, r, hid, kb),
        grid=(4, nblk),
        in_specs=[x_spec, sup_spec, full((d_in, hid)), full((2 * hid, 1)),
                  full((hid, hid)), full((1, hid)), full((kb, d_in)),
                  full((hid, d_out)),
                  pl.BlockSpec((1, 1, r), lambda s, i: (i, 0, 0)),
                  full((c, c))],
        out_specs=pl.BlockSpec(
            (r, d_out), lambda s, i: (jnp.where(s == 3, i, 0), 0)),
        out_shape=jax.ShapeDtypeStruct((n, d_out), _F32),
        scratch_shapes=[pltpu.VMEM((n, hid), _BF16),
                        pltpu.VMEM((n, 1), _F32),
                        pltpu.VMEM((1, n), _F32),
                        pltpu.VMEM((n, 1), _F32),
                        pltpu.VMEM((n, hid), _F32),
                        pltpu.VMEM((n, hid), _BF16),
                        pltpu.VMEM((n, hid), _F32),
                        pltpu.VMEM((n, d_out), _BF16),
                        pltpu.VMEM((n, d_out), _BF16),
                        pltpu.VMEM((hid, d_out), _F32),
                        pltpu.VMEM((hid, hid), _F32),
                        pltpu.VMEM((c, hid), _F32),
                        pltpu.VMEM((c, hid), _F32),
                        pltpu.VMEM((n, n), _BF16),
                        pltpu.VMEM((n, 1), _F32),
                        pltpu.VMEM((n, 1), _F32),
                        pltpu.VMEM((n, n), _BF16)],
    )(x, support, W, a, fc0_W, fcb, C_b_prime, gcn_W, y3, Q)
    return res


# speedup vs baseline: 1.1700x; 1.0621x over previous
import functools

import jax
import jax.numpy as jnp
from jax.experimental import pallas as pl
from jax.experimental.pallas import tpu as pltpu

_F32 = jnp.float32
_BF16 = jnp.bfloat16


def _dot(a, b):
    return jnp.dot(a, b, preferred_element_type=_F32)


def _dg(a, b, dims):
    return jax.lax.dot_general(a, b, (dims, ((), ())),
                               preferred_element_type=_F32)


def _body(c, n, r, hid, kb,
          x_ref, sup_ref, w_ref, a_ref, fcw_ref, fcb_ref, cb_ref,
          gcnw_ref, y_ref, q_ref, res_ref,
          whb_s, wh1_s, wh2r_s, lsum_s, f0_s, f0b_s, z0_s, xw_s, outb_s,
          g_s, m_s, cs_s, cnt_s, p_s, sm_s, psq_s, supb_s):
    s = pl.program_id(0)
    i = pl.program_id(1)
    rows = pl.ds(i * r, r)

    @pl.when(s == 0)
    def _prep():
        xb = x_ref[...]
        whb = _dot(xb, w_ref[...])
        whb_s[rows, :] = whb.astype(_BF16)
        a1 = a_ref[:hid, :]
        a2 = a_ref[hid:, :]
        wh1_s[rows, :] = _dot(whb, a1)
        wh2r_s[:, rows] = _dg(a2, whb, (((0,), (1,))))
        f0b = _dg(whb, fcw_ref[...], (((1,), (1,)))) + fcb_ref[...]
        f0_s[rows, :] = f0b
        f0b_s[rows, :] = f0b.astype(_BF16)
        cb = cb_ref[...]
        xsqt = _dg(jnp.ones((1, xb.shape[1]), _F32), xb * xb,
                   (((1,), (1,))))
        cbsqt = jnp.sum(cb * cb, axis=1, keepdims=True)
        crosst = _dg(cb, xb, (((1,), (1,))))
        dt = jnp.sqrt(jnp.maximum(xsqt - 2.0 * crosst + cbsqt, 0.0))
        sx = jnp.exp(-dt) + 1e-10
        lsum_s[:, rows] = (jnp.sum(jnp.log(sx), axis=0, keepdims=True)
                           - kb * jnp.log(jnp.sum(sx, axis=0, keepdims=True)))

    @pl.when(s == 1)
    def _attn():
        sup = sup_ref[...]
        supb_s[rows, :] = sup.astype(_BF16)
        e = wh1_s[rows, :] + wh2r_s[...]
        e = jnp.maximum(e, 0.2 * e)
        p = jnp.where(sup > 0, jnp.exp(e), 0.0)
        pb = p.astype(_BF16)
        p_s[rows, :] = pb
        sm = jnp.sum(p, axis=1, keepdims=True)
        sm_s[rows, :] = sm
        psq_s[rows, :] = jnp.sum(p * p, axis=1, keepdims=True)
        hp = _dot(pb, whb_s[...]) / sm
        z0_s[rows, :] = hp
        gat = jnp.where(hp > 0, hp, jnp.exp(jnp.minimum(hp, 0.0)) - 1.0)
        xw_s[rows, :] = _dot(gat.astype(_BF16),
                             gcnw_ref[...].astype(_BF16)).astype(_BF16)
        yrow = y_ref[0]
        oh = (jax.lax.broadcasted_iota(jnp.int32, (c, r), 0) == yrow)
        oh = oh.astype(_F32)
        csb = _dot(oh, hp)
        cntb = _dot(oh, jnp.ones((r, cs_s.shape[1]), dtype=_F32))

        @pl.when(i == 0)
        def _init():
            cs_s[...] = csb
            cnt_s[...] = cntb

        @pl.when(i > 0)
        def _acc():
            cs_s[...] += csb
            cnt_s[...] += cntb

    @pl.when(s == 2)
    def _spread():
        out_blk = _dot(supb_s[rows, :], xw_s[...])
        outb_s[rows, :] = out_blk.astype(_BF16)
        f0_blk = f0_s[rows, :]
        gb = _dg(f0_blk, out_blk, (((0,), (0,))))
        mb = _dg(f0_blk, f0_blk, (((0,), (0,))))

        @pl.when(i == 0)
        def _init():
            g_s[...] = gb
            m_s[...] = mb

        @pl.when(i > 0)
        def _acc():
            g_s[...] += gb
            m_s[...] += mb

    @pl.when(s == 3)
    def _final():
        ca = cs_s[...] / jnp.maximum(cnt_s[...], 1.0)
        z0 = z0_s[rows, :]
        yrow = y_ref[0]
        oh = (jax.lax.broadcasted_iota(jnp.int32, (c, r), 0) == yrow)
        oh = oh.astype(_F32)
        logq = jnp.log(q_ref[...])
        mu = jnp.sum(cs_s[...], axis=0, keepdims=True) * (1.0 / n)
        zp = z0 - mu
        cap = ca - mu
        zsqt = _dg(jnp.ones((1, zp.shape[1]), _F32), zp * zp,
                   (((1,), (1,))))
        casqt = jnp.sum(cap * cap, axis=1, keepdims=True)
        zct = _dg(cap, zp, (((1,), (1,))))
        dt = jnp.sqrt(jnp.maximum(zsqt - 2.0 * zct + casqt, 0.0))
        skt = jnp.exp(-dt) + 1e-10
        phit = skt / jnp.sum(skt, axis=0, keepdims=True)
        sylqt = _dg(logq, oh, (((0,), (0,))))
        dbt = phit * lsum_s[:, rows] - sylqt
        dsumt = jnp.sum(dbt, axis=0, keepdims=True)
        cmatt = (dsumt * phit - dbt) / (dt + 1e-12)
        csum = _dg(cmatt, jnp.ones((cmatt.shape[0], 1), _F32),
                   (((0,), (0,))))
        v = csum * zp - _dg(cmatt, cap, (((0,), (0,))))
        pb = p_s[rows, :]
        inv_sm = 1.0 / sm_s[rows, :]
        att_f0 = _dot(pb, f0b_s[...]) * inv_sm
        cross = jnp.sum(v * att_f0, axis=1, keepdims=True) * (1.0 / n)
        asq = psq_s[rows, :] * (inv_sm * inv_sm)
        vm = _dot(v, m_s[...])
        gsq = jnp.sum(vm * v, axis=1, keepdims=True) * (1.0 / (n * n))
        rn = jnp.sqrt(asq - 2.0 * cross + gsq)
        pout = _dot(pb, outb_s[...]) * inv_sm
        z1 = (pout - _dot(v, g_s[...]) * (1.0 / n)) / rn
        res_ref[...] = jnp.maximum(z1, 0.0)


def kernel(x, support, y, C_b_prime, Q, W, a, fc0_W, fc0_b, gcn_W):
    n, d_in = x.shape
    hid = W.shape[1]
    d_out = gcn_W.shape[1]
    c = Q.shape[0]
    kb = C_b_prime.shape[0]
    r = 128
    nblk = n // r
    y3 = y.astype(jnp.int32).reshape(nblk, 1, r)
    fcb = fc0_b.reshape(1, hid)

    full = lambda bs: pl.BlockSpec(bs, lambda s, i: (0, 0))
    x_spec = pl.BlockSpec(
        (r, d_in), lambda s, i: (jnp.where(s == 0, i, nblk - 1), 0))
    sup_spec = pl.BlockSpec(
        (r, n),
        lambda s, i: (jnp.where(s == 1, i,
                                jnp.where(s == 0, 0, nblk - 1)), 0))

    res = pl.pallas_call(
        functools.partial(_body, c, n, r, hid, kb),
        grid=(4, nblk),
        in_specs=[x_spec, sup_spec, full((d_in, hid)), full((2 * hid, 1)),
                  full((hid, hid)), full((1, hid)), full((kb, d_in)),
                  full((hid, d_out)),
                  pl.BlockSpec((1, 1, r), lambda s, i: (i, 0, 0)),
                  full((c, c))],
        out_specs=pl.BlockSpec(
            (r, d_out), lambda s, i: (jnp.where(s == 3, i, 0), 0)),
        out_shape=jax.ShapeDtypeStruct((n, d_out), _F32),
        scratch_shapes=[pltpu.VMEM((n, hid), _BF16),
                        pltpu.VMEM((n, 1), _F32),
                        pltpu.VMEM((1, n), _F32),
                        pltpu.VMEM((1, n), _F32),
                        pltpu.VMEM((n, hid), _F32),
                        pltpu.VMEM((n, hid), _BF16),
                        pltpu.VMEM((n, hid), _F32),
                        pltpu.VMEM((n, d_out), _BF16),
                        pltpu.VMEM((n, d_out), _BF16),
                        pltpu.VMEM((hid, d_out), _F32),
                        pltpu.VMEM((hid, hid), _F32),
                        pltpu.VMEM((c, hid), _F32),
                        pltpu.VMEM((c, hid), _F32),
                        pltpu.VMEM((n, n), _BF16),
                        pltpu.VMEM((n, 1), _F32),
                        pltpu.VMEM((n, 1), _F32),
                        pltpu.VMEM((n, n), _BF16)],
    )(x, support, W, a, fc0_W, fcb, C_b_prime, gcn_W, y3, Q)
    return res


# speedup vs baseline: 1.6744x; 1.4312x over previous
import functools

import jax
import jax.numpy as jnp
from jax.experimental import pallas as pl
from jax.experimental.pallas import tpu as pltpu

_F32 = jnp.float32
_BF16 = jnp.bfloat16


def _dot(a, b):
    return jnp.dot(a, b, preferred_element_type=_F32)


def _dg(a, b, dims):
    return jax.lax.dot_general(a, b, (dims, ((), ())),
                               preferred_element_type=_F32)


def _body(c, n, r, hid, kb,
          x_ref, sup_ref, w_ref, a_ref, fcw_ref, fcb_ref, cb_ref,
          gcnw_ref, y_ref, q_ref, res_ref,
          whb_s, wh1_s, wh2r_s, lsum_s, f0_s, f0b_s, z0_s, xw_s, outb_s,
          g_s, m_s, cs_s, cnt_s, p_s, sm_s, psq_s, supb_s):
    s = pl.program_id(0)
    i = pl.program_id(1)
    rows = pl.ds(i * r, r)

    @pl.when(s == 0)
    def _prep():
        xb = x_ref[...]
        whb = _dot(xb, w_ref[...])
        whb_s[rows, :] = whb.astype(_BF16)
        a1 = a_ref[:hid, :]
        a2 = a_ref[hid:, :]
        wh1_s[rows, :] = _dot(whb, a1)
        wh2r_s[:, rows] = _dg(a2, whb, (((0,), (1,))))
        f0b = _dg(whb, fcw_ref[...], (((1,), (1,)))) + fcb_ref[...]
        f0_s[rows, :] = f0b
        f0b_s[rows, :] = f0b.astype(_BF16)
        cb = cb_ref[...]
        xsqt = _dg(jnp.ones((1, xb.shape[1]), _F32), xb * xb,
                   (((1,), (1,))))
        cbsqt = jnp.sum(cb * cb, axis=1, keepdims=True)
        crosst = _dg(cb, xb, (((1,), (1,))))
        dt = jnp.sqrt(jnp.maximum(xsqt - 2.0 * crosst + cbsqt, 0.0))
        sx = jnp.exp(-dt) + 1e-10
        lsum_s[:, rows] = (jnp.sum(jnp.log(sx), axis=0, keepdims=True)
                           - kb * jnp.log(jnp.sum(sx, axis=0, keepdims=True)))

    @pl.when(s == 1)
    def _attn():
        sup = sup_ref[...]
        supb_s[rows, :] = sup.astype(_BF16)
        e = wh1_s[rows, :] + wh2r_s[...]
        e = jnp.maximum(e, 0.2 * e)
        p = jnp.where(sup > 0, jnp.exp(e), 0.0)
        pb = p.astype(_BF16)
        p_s[rows, :] = pb
        sm = jnp.sum(p, axis=1, keepdims=True)
        sm_s[rows, :] = sm
        psq_s[rows, :] = jnp.sum(p * p, axis=1, keepdims=True)
        hp = _dot(pb, whb_s[...]) / sm
        z0_s[rows, :] = hp
        gat = jnp.where(hp > 0, hp, jnp.exp(jnp.minimum(hp, 0.0)) - 1.0)
        xw_s[rows, :] = _dot(gat.astype(_BF16),
                             gcnw_ref[...].astype(_BF16)).astype(_BF16)
        yrow = y_ref[0]
        oh = (jax.lax.broadcasted_iota(jnp.int32, (c, r), 0) == yrow)
        oh = oh.astype(_F32)
        csb = _dot(oh, hp)
        cntb = _dot(oh, jnp.ones((r, cs_s.shape[1]), dtype=_F32))

        @pl.when(i == 0)
        def _init():
            cs_s[...] = csb
            cnt_s[...] = cntb

        @pl.when(i > 0)
        def _acc():
            cs_s[...] += csb
            cnt_s[...] += cntb

    @pl.when(s == 2)
    def _spread():
        out_blk = _dot(supb_s[rows, :], xw_s[...])
        outb_s[rows, :] = out_blk.astype(_BF16)
        f0_blk = f0_s[rows, :]
        gb = _dg(f0_blk, out_blk, (((0,), (0,))))
        mb = _dg(f0_blk, f0_blk, (((0,), (0,))))

        @pl.when(i == 0)
        def _init():
            g_s[...] = gb
            m_s[...] = mb

        @pl.when(i > 0)
        def _acc():
            g_s[...] += gb
            m_s[...] += mb

    @pl.when(s == 3)
    def _final():
        ca = cs_s[...] / jnp.maximum(cnt_s[...], 1.0)
        z0 = z0_s[rows, :]
        yrow = y_ref[0]
        oh = (jax.lax.broadcasted_iota(jnp.int32, (c, r), 0) == yrow)
        oh = oh.astype(_F32)
        logq = jnp.log(q_ref[...])
        mu = jnp.sum(cs_s[...], axis=0, keepdims=True) * (1.0 / n)
        zp = z0 - mu
        cap = ca - mu
        zsqt = _dg(jnp.ones((1, zp.shape[1]), _F32), zp * zp,
                   (((1,), (1,))))
        casqt = jnp.sum(cap * cap, axis=1, keepdims=True)
        zct = _dg(cap, zp, (((1,), (1,))))
        dt = jnp.sqrt(jnp.maximum(zsqt - 2.0 * zct + casqt, 0.0))
        skt = jnp.exp(-dt) + 1e-10
        phit = skt / jnp.sum(skt, axis=0, keepdims=True)
        sylqt = _dg(logq, oh, (((0,), (0,))))
        dbt = phit * lsum_s[:, rows] - sylqt
        dsumt = jnp.sum(dbt, axis=0, keepdims=True)
        cmatt = (dsumt * phit - dbt) / (dt + 1e-12)
        csum = _dg(cmatt, jnp.ones((cmatt.shape[0], 1), _F32),
                   (((0,), (0,))))
        v = csum * zp - _dg(cmatt, cap, (((0,), (0,))))
        pb = p_s[rows, :]
        inv_sm = 1.0 / sm_s[rows, :]
        att_f0 = _dot(pb, f0b_s[...]) * inv_sm
        cross = jnp.sum(v * att_f0, axis=1, keepdims=True) * (1.0 / n)
        asq = psq_s[rows, :] * (inv_sm * inv_sm)
        vm = _dot(v, m_s[...])
        gsq = jnp.sum(vm * v, axis=1, keepdims=True) * (1.0 / (n * n))
        rn = jnp.sqrt(asq - 2.0 * cross + gsq)
        pout = _dot(pb, outb_s[...]) * inv_sm
        z1 = (pout - _dot(v, g_s[...]) * (1.0 / n)) / rn
        res_ref[...] = jnp.maximum(z1, 0.0)


def kernel(x, support, y, C_b_prime, Q, W, a, fc0_W, fc0_b, gcn_W):
    n, d_in = x.shape
    hid = W.shape[1]
    d_out = gcn_W.shape[1]
    c = Q.shape[0]
    kb = C_b_prime.shape[0]
    r = 256
    nblk = n // r
    y3 = y.astype(jnp.int32).reshape(nblk, 1, r)
    fcb = fc0_b.reshape(1, hid)

    full = lambda bs: pl.BlockSpec(bs, lambda s, i: (0, 0))
    x_spec = pl.BlockSpec(
        (r, d_in), lambda s, i: (jnp.where(s == 0, i, nblk - 1), 0))
    sup_spec = pl.BlockSpec(
        (r, n),
        lambda s, i: (jnp.where(s == 1, i,
                                jnp.where(s == 0, 0, nblk - 1)), 0))

    res = pl.pallas_call(
        functools.partial(_body, c, n, r, hid, kb),
        grid=(4, nblk),
        in_specs=[x_spec, sup_spec, full((d_in, hid)), full((2 * hid, 1)),
                  full((hid, hid)), full((1, hid)), full((kb, d_in)),
                  full((hid, d_out)),
                  pl.BlockSpec((1, 1, r), lambda s, i: (i, 0, 0)),
                  full((c, c))],
        out_specs=pl.BlockSpec(
            (r, d_out), lambda s, i: (jnp.where(s == 3, i, 0), 0)),
        out_shape=jax.ShapeDtypeStruct((n, d_out), _F32),
        scratch_shapes=[pltpu.VMEM((n, hid), _BF16),
                        pltpu.VMEM((n, 1), _F32),
                        pltpu.VMEM((1, n), _F32),
                        pltpu.VMEM((1, n), _F32),
                        pltpu.VMEM((n, hid), _F32),
                        pltpu.VMEM((n, hid), _BF16),
                        pltpu.VMEM((n, hid), _F32),
                        pltpu.VMEM((n, d_out), _BF16),
                        pltpu.VMEM((n, d_out), _BF16),
                        pltpu.VMEM((hid, d_out), _F32),
                        pltpu.VMEM((hid, hid), _F32),
                        pltpu.VMEM((c, hid), _F32),
                        pltpu.VMEM((c, hid), _F32),
                        pltpu.VMEM((n, n), _BF16),
                        pltpu.VMEM((n, 1), _F32),
                        pltpu.VMEM((n, 1), _F32),
                        pltpu.VMEM((n, n), _BF16)],
    )(x, support, W, a, fc0_W, fcb, C_b_prime, gcn_W, y3, Q)
    return res


# speedup vs baseline: 1.9199x; 1.1466x over previous
import functools

import jax
import jax.numpy as jnp
from jax.experimental import pallas as pl
from jax.experimental.pallas import tpu as pltpu

_F32 = jnp.float32
_BF16 = jnp.bfloat16


def _dot(a, b):
    return jnp.dot(a, b, preferred_element_type=_F32)


def _dg(a, b, dims):
    return jax.lax.dot_general(a, b, (dims, ((), ())),
                               preferred_element_type=_F32)


def _body(c, n, r, hid, kb,
          x_ref, sup_ref, w_ref, a_ref, fcw_ref, fcb_ref, cb_ref,
          gcnw_ref, y_ref, q_ref, res_ref,
          whb_s, wh1_s, wh2r_s, lsum_s, f0_s, f0b_s, z0_s, xw_s, outb_s,
          g_s, m_s, cs_s, cnt_s, p_s, sm_s, psq_s, supb_s):
    s = pl.program_id(0)
    i = pl.program_id(1)
    rows = pl.ds(i * r, r)

    @pl.when(s == 0)
    def _prep():
        xb = x_ref[...]
        whb = _dot(xb, w_ref[...])
        whb_s[rows, :] = whb.astype(_BF16)
        a1 = a_ref[:hid, :]
        a2 = a_ref[hid:, :]
        wh1_s[rows, :] = _dot(whb, a1)
        wh2r_s[:, rows] = _dg(a2, whb, (((0,), (1,))))
        f0b = _dg(whb, fcw_ref[...], (((1,), (1,)))) + fcb_ref[...]
        f0_s[rows, :] = f0b
        f0b_s[rows, :] = f0b.astype(_BF16)
        cb = cb_ref[...]
        xsqt = _dg(jnp.ones((1, xb.shape[1]), _F32), xb * xb,
                   (((1,), (1,))))
        cbsqt = jnp.sum(cb * cb, axis=1, keepdims=True)
        crosst = _dg(cb, xb, (((1,), (1,))))
        dt = jnp.sqrt(jnp.maximum(xsqt - 2.0 * crosst + cbsqt, 0.0))
        sx = jnp.exp(-dt) + 1e-10
        lsum_s[:, rows] = (jnp.sum(jnp.log(sx), axis=0, keepdims=True)
                           - kb * jnp.log(jnp.sum(sx, axis=0, keepdims=True)))

    @pl.when(s == 1)
    def _attn():
        sup = sup_ref[...]
        supb_s[rows, :] = sup.astype(_BF16)
        e = wh1_s[rows, :] + wh2r_s[...]
        e = jnp.maximum(e, 0.2 * e)
        p = jnp.where(sup > 0, jnp.exp(e), 0.0)
        pb = p.astype(_BF16)
        p_s[rows, :] = pb
        sm = jnp.sum(p, axis=1, keepdims=True)
        sm_s[rows, :] = sm
        psq_s[rows, :] = jnp.sum(p * p, axis=1, keepdims=True)
        hp = _dot(pb, whb_s[...]) / sm
        z0_s[rows, :] = hp
        gat = jnp.where(hp > 0, hp, jnp.exp(jnp.minimum(hp, 0.0)) - 1.0)
        xw_s[rows, :] = _dot(gat.astype(_BF16),
                             gcnw_ref[...].astype(_BF16)).astype(_BF16)
        yrow = y_ref[0]
        oh = (jax.lax.broadcasted_iota(jnp.int32, (c, r), 0) == yrow)
        oh = oh.astype(_F32)
        csb = _dot(oh, hp)
        cntb = _dot(oh, jnp.ones((r, cs_s.shape[1]), dtype=_F32))

        @pl.when(i == 0)
        def _init():
            cs_s[...] = csb
            cnt_s[...] = cntb

        @pl.when(i > 0)
        def _acc():
            cs_s[...] += csb
            cnt_s[...] += cntb

    @pl.when(s == 2)
    def _spread():
        out_blk = _dot(supb_s[rows, :], xw_s[...])
        outb_s[rows, :] = out_blk.astype(_BF16)
        f0_blk = f0_s[rows, :]
        gb = _dg(f0_blk, out_blk, (((0,), (0,))))
        mb = _dg(f0_blk, f0_blk, (((0,), (0,))))

        @pl.when(i == 0)
        def _init():
            g_s[...] = gb
            m_s[...] = mb

        @pl.when(i > 0)
        def _acc():
            g_s[...] += gb
            m_s[...] += mb

    @pl.when(s == 3)
    def _final():
        ca = cs_s[...] / jnp.maximum(cnt_s[...], 1.0)
        z0 = z0_s[rows, :]
        yrow = y_ref[0]
        oh = (jax.lax.broadcasted_iota(jnp.int32, (c, r), 0) == yrow)
        oh = oh.astype(_F32)
        logq = jnp.log(q_ref[...])
        mu = jnp.sum(cs_s[...], axis=0, keepdims=True) * (1.0 / n)
        zp = z0 - mu
        cap = ca - mu
        zsqt = _dg(jnp.ones((1, zp.shape[1]), _F32), zp * zp,
                   (((1,), (1,))))
        casqt = jnp.sum(cap * cap, axis=1, keepdims=True)
        zct = _dg(cap, zp, (((1,), (1,))))
        dt = jnp.sqrt(jnp.maximum(zsqt - 2.0 * zct + casqt, 0.0))
        skt = jnp.exp(-dt) + 1e-10
        phit = skt / jnp.sum(skt, axis=0, keepdims=True)
        sylqt = _dg(logq, oh, (((0,), (0,))))
        dbt = phit * lsum_s[:, rows] - sylqt
        dsumt = jnp.sum(dbt, axis=0, keepdims=True)
        cmatt = (dsumt * phit - dbt) / (dt + 1e-12)
        csum = _dg(cmatt, jnp.ones((cmatt.shape[0], 1), _F32),
                   (((0,), (0,))))
        v = csum * zp - _dg(cmatt, cap, (((0,), (0,))))
        pb = p_s[rows, :]
        inv_sm = 1.0 / sm_s[rows, :]
        att_f0 = _dot(pb, f0b_s[...]) * inv_sm
        cross = jnp.sum(v * att_f0, axis=1, keepdims=True) * (1.0 / n)
        asq = psq_s[rows, :] * (inv_sm * inv_sm)
        vm = _dot(v, m_s[...])
        gsq = jnp.sum(vm * v, axis=1, keepdims=True) * (1.0 / (n * n))
        rn = jnp.sqrt(asq - 2.0 * cross + gsq)
        pout = _dot(pb, outb_s[...]) * inv_sm
        z1 = (pout - _dot(v, g_s[...]) * (1.0 / n)) / rn
        res_ref[...] = jnp.maximum(z1, 0.0)


def kernel(x, support, y, C_b_prime, Q, W, a, fc0_W, fc0_b, gcn_W):
    n, d_in = x.shape
    hid = W.shape[1]
    d_out = gcn_W.shape[1]
    c = Q.shape[0]
    kb = C_b_prime.shape[0]
    r = 512
    nblk = n // r
    y3 = y.astype(jnp.int32).reshape(nblk, 1, r)
    fcb = fc0_b.reshape(1, hid)

    full = lambda bs: pl.BlockSpec(bs, lambda s, i: (0, 0))
    x_spec = pl.BlockSpec(
        (r, d_in), lambda s, i: (jnp.where(s == 0, i, nblk - 1), 0))
    sup_spec = pl.BlockSpec(
        (r, n),
        lambda s, i: (jnp.where(s == 1, i,
                                jnp.where(s == 0, 0, nblk - 1)), 0))

    res = pl.pallas_call(
        functools.partial(_body, c, n, r, hid, kb),
        grid=(4, nblk),
        in_specs=[x_spec, sup_spec, full((d_in, hid)), full((2 * hid, 1)),
                  full((hid, hid)), full((1, hid)), full((kb, d_in)),
                  full((hid, d_out)),
                  pl.BlockSpec((1, 1, r), lambda s, i: (i, 0, 0)),
                  full((c, c))],
        out_specs=pl.BlockSpec(
            (r, d_out), lambda s, i: (jnp.where(s == 3, i, 0), 0)),
        out_shape=jax.ShapeDtypeStruct((n, d_out), _F32),
        scratch_shapes=[pltpu.VMEM((n, hid), _BF16),
                        pltpu.VMEM((n, 1), _F32),
                        pltpu.VMEM((1, n), _F32),
                        pltpu.VMEM((1, n), _F32),
                        pltpu.VMEM((n, hid), _F32),
                        pltpu.VMEM((n, hid), _BF16),
                        pltpu.VMEM((n, hid), _F32),
                        pltpu.VMEM((n, d_out), _BF16),
                        pltpu.VMEM((n, d_out), _BF16),
                        pltpu.VMEM((hid, d_out), _F32),
                        pltpu.VMEM((hid, hid), _F32),
                        pltpu.VMEM((c, hid), _F32),
                        pltpu.VMEM((c, hid), _F32),
                        pltpu.VMEM((n, n), _BF16),
                        pltpu.VMEM((n, 1), _F32),
                        pltpu.VMEM((n, 1), _F32),
                        pltpu.VMEM((n, n), _BF16)],
    )(x, support, W, a, fc0_W, fcb, C_b_prime, gcn_W, y3, Q)
    return res


# speedup vs baseline: 2.0664x; 1.0763x over previous
import functools

import jax
import jax.numpy as jnp
from jax.experimental import pallas as pl
from jax.experimental.pallas import tpu as pltpu

_F32 = jnp.float32
_BF16 = jnp.bfloat16


def _dot(a, b):
    return jnp.dot(a, b, preferred_element_type=_F32)


def _dg(a, b, dims):
    return jax.lax.dot_general(a, b, (dims, ((), ())),
                               preferred_element_type=_F32)


def _body(c, n, r, hid, kb,
          x_ref, sup_ref, w_ref, a_ref, fcw_ref, fcb_ref, cb_ref,
          gcnw_ref, y_ref, q_ref, res_ref,
          whf_s, wh1_s, wh2r_s, lsum_s, f0_s, af_s, z0_s, xw_s, outb_s,
          g_s, m_s, cs_s, cnt_s, p_s, sm_s, psq_s, supb_s):
    s = pl.program_id(0)
    i = pl.program_id(1)
    rows = pl.ds(i * r, r)

    @pl.when(s == 0)
    def _prep():
        xb = x_ref[...]
        whb = _dot(xb, w_ref[...])
        whf_s[rows, :hid] = whb.astype(_BF16)
        a1 = a_ref[:hid, :]
        a2 = a_ref[hid:, :]
        wh1_s[rows, :] = _dot(whb, a1)
        wh2r_s[:, rows] = _dg(a2, whb, (((0,), (1,))))
        f0b = _dg(whb, fcw_ref[...], (((1,), (1,)))) + fcb_ref[...]
        f0_s[rows, :] = f0b
        whf_s[rows, hid:] = f0b.astype(_BF16)
        cb = cb_ref[...]
        xsqt = _dg(jnp.ones((1, xb.shape[1]), _F32), xb * xb,
                   (((1,), (1,))))
        cbsqt = jnp.sum(cb * cb, axis=1, keepdims=True)
        crosst = _dg(cb, xb, (((1,), (1,))))
        dt = jnp.sqrt(jnp.maximum(xsqt - 2.0 * crosst + cbsqt, 0.0))
        sx = jnp.exp(-dt) + 1e-10
        lsum_s[:, rows] = (jnp.sum(jnp.log(sx), axis=0, keepdims=True)
                           - kb * jnp.log(jnp.sum(sx, axis=0, keepdims=True)))

    @pl.when(s == 1)
    def _attn():
        sup = sup_ref[...]
        supb_s[rows, :] = sup.astype(_BF16)
        e = wh1_s[rows, :] + wh2r_s[...]
        e = jnp.maximum(e, 0.2 * e)
        p = jnp.where(sup > 0, jnp.exp(e), 0.0)
        pb = p.astype(_BF16)
        p_s[rows, :] = pb
        sm = jnp.sum(p, axis=1, keepdims=True)
        sm_s[rows, :] = sm
        psq_s[rows, :] = jnp.sum(p * p, axis=1, keepdims=True)
        hpaf = _dot(pb, whf_s[...])
        hp = hpaf[:, :hid] / sm
        af_s[rows, :] = hpaf[:, hid:]
        z0_s[rows, :] = hp
        gat = jnp.where(hp > 0, hp, jnp.exp(jnp.minimum(hp, 0.0)) - 1.0)
        xw_s[rows, :] = _dot(gat.astype(_BF16),
                             gcnw_ref[...].astype(_BF16)).astype(_BF16)
        yrow = y_ref[0]
        oh = (jax.lax.broadcasted_iota(jnp.int32, (c, r), 0) == yrow)
        oh = oh.astype(_F32)
        csb = _dot(oh, hp)
        cntb = _dot(oh, jnp.ones((r, cs_s.shape[1]), dtype=_F32))

        @pl.when(i == 0)
        def _init():
            cs_s[...] = csb
            cnt_s[...] = cntb

        @pl.when(i > 0)
        def _acc():
            cs_s[...] += csb
            cnt_s[...] += cntb

    @pl.when(s == 2)
    def _spread():
        out_blk = _dot(supb_s[rows, :], xw_s[...])
        outb_s[rows, :] = out_blk.astype(_BF16)
        f0_blk = f0_s[rows, :]
        gb = _dg(f0_blk, out_blk, (((0,), (0,))))
        mb = _dg(f0_blk, f0_blk, (((0,), (0,))))

        @pl.when(i == 0)
        def _init():
            g_s[...] = gb
            m_s[...] = mb

        @pl.when(i > 0)
        def _acc():
            g_s[...] += gb
            m_s[...] += mb

    @pl.when(s == 3)
    def _final():
        ca = cs_s[...] / jnp.maximum(cnt_s[...], 1.0)
        z0 = z0_s[rows, :]
        yrow = y_ref[0]
        oh = (jax.lax.broadcasted_iota(jnp.int32, (c, r), 0) == yrow)
        oh = oh.astype(_F32)
        logq = jnp.log(q_ref[...])
        mu = jnp.sum(cs_s[...], axis=0, keepdims=True) * (1.0 / n)
        zp = z0 - mu
        cap = ca - mu
        zsqt = _dg(jnp.ones((1, zp.shape[1]), _F32), zp * zp,
                   (((1,), (1,))))
        casqt = jnp.sum(cap * cap, axis=1, keepdims=True)
        zct = _dg(cap, zp, (((1,), (1,))))
        dt = jnp.sqrt(jnp.maximum(zsqt - 2.0 * zct + casqt, 0.0))
        skt = jnp.exp(-dt) + 1e-10
        phit = skt / jnp.sum(skt, axis=0, keepdims=True)
        sylqt = _dg(logq, oh, (((0,), (0,))))
        dbt = phit * lsum_s[:, rows] - sylqt
        dsumt = jnp.sum(dbt, axis=0, keepdims=True)
        cmatt = (dsumt * phit - dbt) / (dt + 1e-12)
        csum = _dg(cmatt, jnp.ones((cmatt.shape[0], 1), _F32),
                   (((0,), (0,))))
        v = csum * zp - _dg(cmatt, cap, (((0,), (0,))))
        pb = p_s[rows, :]
        inv_sm = 1.0 / sm_s[rows, :]
        att_f0 = af_s[rows, :] * inv_sm
        cross = jnp.sum(v * att_f0, axis=1, keepdims=True) * (1.0 / n)
        asq = psq_s[rows, :] * (inv_sm * inv_sm)
        vm = _dot(v, m_s[...])
        gsq = jnp.sum(vm * v, axis=1, keepdims=True) * (1.0 / (n * n))
        rn = jnp.sqrt(asq - 2.0 * cross + gsq)
        pout = _dot(pb, outb_s[...]) * inv_sm
        z1 = (pout - _dot(v, g_s[...]) * (1.0 / n)) / rn
        res_ref[...] = jnp.maximum(z1, 0.0)


def kernel(x, support, y, C_b_prime, Q, W, a, fc0_W, fc0_b, gcn_W):
    n, d_in = x.shape
    hid = W.shape[1]
    d_out = gcn_W.shape[1]
    c = Q.shape[0]
    kb = C_b_prime.shape[0]
    r = 512
    nblk = n // r
    y3 = y.astype(jnp.int32).reshape(nblk, 1, r)
    fcb = fc0_b.reshape(1, hid)

    full = lambda bs: pl.BlockSpec(bs, lambda s, i: (0, 0))
    x_spec = pl.BlockSpec(
        (r, d_in), lambda s, i: (jnp.where(s == 0, i, nblk - 1), 0))
    sup_spec = pl.BlockSpec(
        (r, n),
        lambda s, i: (jnp.where(s == 1, i,
                                jnp.where(s == 0, 0, nblk - 1)), 0))

    res = pl.pallas_call(
        functools.partial(_body, c, n, r, hid, kb),
        grid=(4, nblk),
        in_specs=[x_spec, sup_spec, full((d_in, hid)), full((2 * hid, 1)),
                  full((hid, hid)), full((1, hid)), full((kb, d_in)),
                  full((hid, d_out)),
                  pl.BlockSpec((1, 1, r), lambda s, i: (i, 0, 0)),
                  full((c, c))],
        out_specs=pl.BlockSpec(
            (r, d_out), lambda s, i: (jnp.where(s == 3, i, 0), 0)),
        out_shape=jax.ShapeDtypeStruct((n, d_out), _F32),
        scratch_shapes=[pltpu.VMEM((n, 2 * hid), _BF16),
                        pltpu.VMEM((n, 1), _F32),
                        pltpu.VMEM((1, n), _F32),
                        pltpu.VMEM((1, n), _F32),
                        pltpu.VMEM((n, hid), _F32),
                        pltpu.VMEM((n, hid), _F32),
                        pltpu.VMEM((n, hid), _F32),
                        pltpu.VMEM((n, d_out), _BF16),
                        pltpu.VMEM((n, d_out), _BF16),
                        pltpu.VMEM((hid, d_out), _F32),
                        pltpu.VMEM((hid, hid), _F32),
                        pltpu.VMEM((c, hid), _F32),
                        pltpu.VMEM((c, hid), _F32),
                        pltpu.VMEM((n, n), _BF16),
                        pltpu.VMEM((n, 1), _F32),
                        pltpu.VMEM((n, 1), _F32),
                        pltpu.VMEM((n, n), _BF16)],
    )(x, support, W, a, fc0_W, fcb, C_b_prime, gcn_W, y3, Q)
    return res


# speedup vs baseline: 2.0675x; 1.0005x over previous
import functools

import jax
import jax.numpy as jnp
from jax.experimental import pallas as pl
from jax.experimental.pallas import tpu as pltpu

_F32 = jnp.float32
_BF16 = jnp.bfloat16


def _dot(a, b):
    return jnp.dot(a, b, preferred_element_type=_F32)


def _dg(a, b, dims):
    return jax.lax.dot_general(a, b, (dims, ((), ())),
                               preferred_element_type=_F32)


def _body(c, n, r, hid, kb,
          x_ref, sup_ref, w_ref, a_ref, fcw_ref, fcb_ref, cb_ref,
          gcnw_ref, y_ref, q_ref, res_ref,
          whf_s, wh1_s, wh2r_s, lsum_s, f0_s, af_s, z0_s, xw_s, outb_s,
          g_s, m_s, cs_s, cnt_s, p_s, sm_s, psq_s, supb_s):
    s = pl.program_id(0)
    i = pl.program_id(1)
    rows = pl.ds(i * r, r)

    @pl.when(s == 0)
    def _prep():
        xb = x_ref[...]
        whb = _dot(xb.astype(_BF16), w_ref[...].astype(_BF16))
        whf_s[rows, :hid] = whb.astype(_BF16)
        a1 = a_ref[:hid, :]
        a2 = a_ref[hid:, :]
        wh1_s[rows, :] = _dot(whb, a1)
        wh2r_s[:, rows] = _dg(a2, whb, (((0,), (1,))))
        f0b = _dg(whb, fcw_ref[...], (((1,), (1,)))) + fcb_ref[...]
        f0_s[rows, :] = f0b
        whf_s[rows, hid:] = f0b.astype(_BF16)
        cb = cb_ref[...]
        xsqt = _dg(jnp.ones((1, xb.shape[1]), _F32), xb * xb,
                   (((1,), (1,))))
        cbsqt = jnp.sum(cb * cb, axis=1, keepdims=True)
        crosst = _dg(cb, xb, (((1,), (1,))))
        dt = jnp.sqrt(jnp.maximum(xsqt - 2.0 * crosst + cbsqt, 0.0))
        sx = jnp.exp(-dt) + 1e-10
        lsum_s[:, rows] = (jnp.sum(jnp.log(sx), axis=0, keepdims=True)
                           - kb * jnp.log(jnp.sum(sx, axis=0, keepdims=True)))

    @pl.when(s == 1)
    def _attn():
        sup = sup_ref[...]
        supb_s[rows, :] = sup.astype(_BF16)
        e = wh1_s[rows, :] + wh2r_s[...]
        e = jnp.maximum(e, 0.2 * e)
        p = jnp.where(sup > 0, jnp.exp(e), 0.0)
        pb = p.astype(_BF16)
        p_s[rows, :] = pb
        sm = jnp.sum(p, axis=1, keepdims=True)
        sm_s[rows, :] = sm
        psq_s[rows, :] = jnp.sum(p * p, axis=1, keepdims=True)
        hpaf = _dot(pb, whf_s[...])
        hp = hpaf[:, :hid] / sm
        af_s[rows, :] = hpaf[:, hid:]
        z0_s[rows, :] = hp
        gat = jnp.where(hp > 0, hp, jnp.exp(jnp.minimum(hp, 0.0)) - 1.0)
        xw_s[rows, :] = _dot(gat.astype(_BF16),
                             gcnw_ref[...].astype(_BF16)).astype(_BF16)
        yrow = y_ref[0]
        oh = (jax.lax.broadcasted_iota(jnp.int32, (c, r), 0) == yrow)
        oh = oh.astype(_F32)
        csb = _dot(oh, hp)
        cntb = _dot(oh, jnp.ones((r, cs_s.shape[1]), dtype=_F32))

        @pl.when(i == 0)
        def _init():
            cs_s[...] = csb
            cnt_s[...] = cntb

        @pl.when(i > 0)
        def _acc():
            cs_s[...] += csb
            cnt_s[...] += cntb

    @pl.when(s == 2)
    def _spread():
        out_blk = _dot(supb_s[rows, :], xw_s[...])
        outb_s[rows, :] = out_blk.astype(_BF16)
        f0_blk = f0_s[rows, :]
        gb = _dg(f0_blk, out_blk, (((0,), (0,))))
        mb = _dg(f0_blk, f0_blk, (((0,), (0,))))

        @pl.when(i == 0)
        def _init():
            g_s[...] = gb
            m_s[...] = mb

        @pl.when(i > 0)
        def _acc():
            g_s[...] += gb
            m_s[...] += mb

    @pl.when(s == 3)
    def _final():
        ca = cs_s[...] / jnp.maximum(cnt_s[...], 1.0)
        z0 = z0_s[rows, :]
        yrow = y_ref[0]
        oh = (jax.lax.broadcasted_iota(jnp.int32, (c, r), 0) == yrow)
        oh = oh.astype(_F32)
        logq = jnp.log(q_ref[...])
        mu = jnp.sum(cs_s[...], axis=0, keepdims=True) * (1.0 / n)
        zp = z0 - mu
        cap = ca - mu
        zsqt = _dg(jnp.ones((1, zp.shape[1]), _F32), zp * zp,
                   (((1,), (1,))))
        casqt = jnp.sum(cap * cap, axis=1, keepdims=True)
        zct = _dg(cap, zp, (((1,), (1,))))
        dt = jnp.sqrt(jnp.maximum(zsqt - 2.0 * zct + casqt, 0.0))
        skt = jnp.exp(-dt) + 1e-10
        phit = skt / jnp.sum(skt, axis=0, keepdims=True)
        sylqt = _dg(logq, oh, (((0,), (0,))))
        dbt = phit * lsum_s[:, rows] - sylqt
        dsumt = jnp.sum(dbt, axis=0, keepdims=True)
        cmatt = (dsumt * phit - dbt) / (dt + 1e-12)
        csum = _dg(cmatt, jnp.ones((cmatt.shape[0], 1), _F32),
                   (((0,), (0,))))
        v = csum * zp - _dg(cmatt, cap, (((0,), (0,))))
        pb = p_s[rows, :]
        inv_sm = 1.0 / sm_s[rows, :]
        att_f0 = af_s[rows, :] * inv_sm
        cross = jnp.sum(v * att_f0, axis=1, keepdims=True) * (1.0 / n)
        asq = psq_s[rows, :] * (inv_sm * inv_sm)
        vm = _dot(v, m_s[...])
        gsq = jnp.sum(vm * v, axis=1, keepdims=True) * (1.0 / (n * n))
        rn = jnp.sqrt(asq - 2.0 * cross + gsq)
        pout = _dot(pb, outb_s[...]) * inv_sm
        z1 = (pout - _dot(v, g_s[...]) * (1.0 / n)) / rn
        res_ref[...] = jnp.maximum(z1, 0.0)


def kernel(x, support, y, C_b_prime, Q, W, a, fc0_W, fc0_b, gcn_W):
    n, d_in = x.shape
    hid = W.shape[1]
    d_out = gcn_W.shape[1]
    c = Q.shape[0]
    kb = C_b_prime.shape[0]
    r = 512
    nblk = n // r
    y3 = y.astype(jnp.int32).reshape(nblk, 1, r)
    fcb = fc0_b.reshape(1, hid)

    full = lambda bs: pl.BlockSpec(bs, lambda s, i: (0, 0))
    x_spec = pl.BlockSpec(
        (r, d_in), lambda s, i: (jnp.where(s == 0, i, nblk - 1), 0))
    sup_spec = pl.BlockSpec(
        (r, n),
        lambda s, i: (jnp.where(s == 1, i,
                                jnp.where(s == 0, 0, nblk - 1)), 0))

    res = pl.pallas_call(
        functools.partial(_body, c, n, r, hid, kb),
        grid=(4, nblk),
        in_specs=[x_spec, sup_spec, full((d_in, hid)), full((2 * hid, 1)),
                  full((hid, hid)), full((1, hid)), full((kb, d_in)),
                  full((hid, d_out)),
                  pl.BlockSpec((1, 1, r), lambda s, i: (i, 0, 0)),
                  full((c, c))],
        out_specs=pl.BlockSpec(
            (r, d_out), lambda s, i: (jnp.where(s == 3, i, 0), 0)),
        out_shape=jax.ShapeDtypeStruct((n, d_out), _F32),
        scratch_shapes=[pltpu.VMEM((n, 2 * hid), _BF16),
                        pltpu.VMEM((n, 1), _F32),
                        pltpu.VMEM((1, n), _F32),
                        pltpu.VMEM((1, n), _F32),
                        pltpu.VMEM((n, hid), _F32),
                        pltpu.VMEM((n, hid), _F32),
                        pltpu.VMEM((n, hid), _F32),
                        pltpu.VMEM((n, d_out), _BF16),
                        pltpu.VMEM((n, d_out), _BF16),
                        pltpu.VMEM((hid, d_out), _F32),
                        pltpu.VMEM((hid, hid), _F32),
                        pltpu.VMEM((c, hid), _F32),
                        pltpu.VMEM((c, hid), _F32),
                        pltpu.VMEM((n, n), _BF16),
                        pltpu.VMEM((n, 1), _F32),
                        pltpu.VMEM((n, 1), _F32),
                        pltpu.VMEM((n, n), _BF16)],
    )(x, support, W, a, fc0_W, fcb, C_b_prime, gcn_W, y3, Q)
    return res
